# baseline (speedup 1.0000x reference)
import jax
import jax.numpy as jnp
from jax import lax
from jax.experimental import pallas as pl
from jax.experimental.pallas import tpu as pltpu

B = 16
P = 1024
K = 20


def _cloud_body(pos_ref, wu_ref, wv_ref, b1_ref, g1_ref, be1_ref,
                w2_ref, b2_ref,
                wcc_ref, wcd_ref, bc2_ref, wla_ref, wlb_ref, bl_ref,
                out_ref, d2_s, ux_s, x1_s, dhi_s, dlo_s, maxd_s):
    f32 = jnp.float32
    HI = lax.Precision.HIGHEST
    x = pos_ref[0]
    sq = jnp.sum(x * x, axis=1, keepdims=True)
    ones = jnp.ones((P, 1), f32)

    g = lax.dot_general(x, x, (((1,), (1,)), ((), ())),
                        preferred_element_type=f32)
    sqrow = lax.dot_general(ones, sq, (((1,), (1,)), ((), ())),
                            preferred_element_type=f32, precision=HI)
    d2_s[...] = (sq + sqrow) - 2.0 * g

    ux_s[...] = jnp.dot(x, wu_ref[...], preferred_element_type=f32)
    x1_s[...] = jnp.full((P, 64), -jnp.inf, f32)

    bf16 = jnp.bfloat16
    xhi = x.astype(bf16)
    r1 = x - xhi.astype(f32)
    xmid = r1.astype(bf16)
    xlo = (r1 - xmid.astype(f32)).astype(bf16)

    iota_j = lax.broadcasted_iota(jnp.int32, (P, P), 1)

    def knn_step(m):
        cur = d2_s[...]
        jidx = jnp.min(jnp.where(cur == m, iota_j, P), axis=1,
                       keepdims=True)
        onehot = iota_j == jidx
        knocked = jnp.where(onehot, jnp.inf, cur)
        d2_s[...] = knocked
        return onehot, jnp.min(knocked, axis=1, keepdims=True)

    def body1(t, m):
        onehot, m = knn_step(m)
        ohb = onehot.astype(bf16)
        dn = (((1,), (0,)), ((), ()))
        xj = (lax.dot_general(ohb, xhi, dn, preferred_element_type=f32)
              + lax.dot_general(ohb, xmid, dn, preferred_element_type=f32)
              + lax.dot_general(ohb, xlo, dn, preferred_element_type=f32))
        a = jnp.dot(xj - x, wv_ref[...], preferred_element_type=f32)
        pre = ux_s[...] + a + b1_ref[...]
        bn = pre / jnp.sqrt(1.0 + 1e-5) * g1_ref[...] + be1_ref[...]
        h = jnp.dot(jax.nn.relu(bn), w2_ref[...],
                    preferred_element_type=f32) + b2_ref[...]
        x1_s[...] = jnp.maximum(x1_s[...], h)
        return m

    m = jnp.min(d2_s[...], axis=1, keepdims=True)
    for t in range(K):
        m = body1(t, m)

    x1 = x1_s[...]
    sq2 = jnp.sum(x1 * x1, axis=1, keepdims=True)
    g2 = lax.dot_general(x1, x1, (((1,), (1,)), ((), ())),
                         preferred_element_type=f32)
    sqrow2 = lax.dot_general(ones, sq2, (((1,), (1,)), ((), ())),
                             preferred_element_type=f32, precision=HI)
    d2_s[...] = (sq2 + sqrow2) - 2.0 * g2

    d = jnp.dot(x1, wcd_ref[...], preferred_element_type=f32)
    bf16 = jnp.bfloat16
    dhi_s[...] = d.astype(bf16)
    dlo_s[...] = (d - dhi_s[...].astype(f32)).astype(bf16)
    maxd_s[...] = jnp.full((P, 128), -jnp.inf, f32)

    def body2(t, m):
        onehot, m = knn_step(m)
        ohb = onehot.astype(bf16)
        dj = (lax.dot_general(ohb, dhi_s[...], (((1,), (0,)), ((), ())),
                              preferred_element_type=f32)
              + lax.dot_general(ohb, dlo_s[...], (((1,), (0,)), ((), ())),
                                preferred_element_type=f32))
        maxd_s[...] = jnp.maximum(maxd_s[...], dj)
        return m

    m = jnp.min(d2_s[...], axis=1, keepdims=True)
    for t in range(K):
        m = body2(t, m)

    x2 = (jnp.dot(x1, wcc_ref[...], preferred_element_type=f32)
          + maxd_s[...] + bc2_ref[...])
    h = (jnp.dot(x1, wla_ref[...], preferred_element_type=f32)
         + jnp.dot(x2, wlb_ref[...], preferred_element_type=f32)
         + bl_ref[...])
    out_ref[0] = jnp.max(h, axis=0, keepdims=True)


def _full(shape):
    return pl.BlockSpec(shape, lambda b: (0,) * len(shape))


def _run(pos_p, wu8, wv8, b1r, g1r, be1r, w2, b2, wcc, wcd, bc2r,
         wla, wlb, blr):
    return pl.pallas_call(
        _cloud_body,
        grid=(B,),
        in_specs=[
            pl.BlockSpec((1, P, 8), lambda b: (b, 0, 0)),
            _full((8, 64)), _full((8, 64)),
            _full((1, 64)), _full((1, 64)), _full((1, 64)),
            _full((64, 64)), _full((1, 64)),
            _full((64, 128)), _full((64, 128)), _full((1, 128)),
            _full((64, 128)), _full((128, 128)), _full((1, 128)),
        ],
        out_specs=pl.BlockSpec((1, 1, 128), lambda b: (b, 0, 0)),
        out_shape=jax.ShapeDtypeStruct((B, 1, 128), jnp.float32),
        scratch_shapes=[
            pltpu.VMEM((P, P), jnp.float32),
            pltpu.VMEM((P, 64), jnp.float32),
            pltpu.VMEM((P, 64), jnp.float32),
            pltpu.VMEM((P, 128), jnp.bfloat16),
            pltpu.VMEM((P, 128), jnp.bfloat16),
            pltpu.VMEM((P, 128), jnp.float32),
        ],
        compiler_params=pltpu.CompilerParams(
            dimension_semantics=("arbitrary",),
        ),
    )(pos_p, wu8, wv8, b1r, g1r, be1r, w2, b2, wcc, wcd, bc2r,
      wla, wlb, blr)


def kernel(pos, batch, W1a, b1a, g1a, be1a, W2a, b2a, Wc2, bc2, Wl, bl):
    f32 = jnp.float32
    wu8 = jnp.zeros((8, 64), f32).at[:3].set(W1a[:3])
    wv8 = jnp.zeros((8, 64), f32).at[:3].set(W1a[3:6])

    wcc = Wc2[:64] - Wc2[64:]
    wcd = Wc2[64:]

    pos_p = jnp.zeros((B, P, 8), f32).at[:, :, :3].set(pos.reshape(B, P, 3))

    out = _run(pos_p, wu8, wv8,
               b1a.reshape(1, 64), g1a.reshape(1, 64), be1a.reshape(1, 64),
               W2a, b2a.reshape(1, 64),
               wcc, wcd, bc2.reshape(1, 128),
               Wl[:64], Wl[64:], bl.reshape(1, 128))
    return out.reshape(B, 128)


# speedup vs baseline: 1.1918x; 1.1918x over previous
import functools

import jax
import jax.numpy as jnp
from jax import lax
from jax.experimental import pallas as pl
from jax.experimental.pallas import tpu as pltpu
from jax.experimental.pallas import tpu_sc as plsc

B = 16
P = 1024
K = 20
BP = B * P
G = 4
PTS_PER_W = BP // 32


def _cloud_body(pos_ref, wu_ref, wv_ref, b1_ref, g1_ref, be1_ref,
                w2_ref, b2_ref, wcc_ref, wcd_ref, wla_ref,
                idx_ref, d_ref, c_ref, hp_ref,
                d2_s, ux_s, x1_s):
    f32 = jnp.float32
    HI = lax.Precision.HIGHEST
    x = pos_ref[0]
    sq = jnp.sum(x * x, axis=1, keepdims=True)
    ones = jnp.ones((P, 1), f32)

    g = lax.dot_general(x, x, (((1,), (1,)), ((), ())),
                        preferred_element_type=f32)
    sqrow = lax.dot_general(ones, sq, (((1,), (1,)), ((), ())),
                            preferred_element_type=f32, precision=HI)
    d2_s[...] = (sq + sqrow) - 2.0 * g

    ux_s[...] = jnp.dot(x, wu_ref[...], preferred_element_type=f32)
    x1_s[...] = jnp.full((P, 64), -jnp.inf, f32)

    bf16 = jnp.bfloat16
    xhi = x.astype(bf16)
    r1 = x - xhi.astype(f32)
    xmid = r1.astype(bf16)
    xlo = (r1 - xmid.astype(f32)).astype(bf16)

    iota_j = lax.broadcasted_iota(jnp.int32, (P, P), 1)

    def knn_step(m):
        cur = d2_s[...]
        jidx = jnp.min(jnp.where(cur == m, iota_j, P), axis=1,
                       keepdims=True)
        onehot = iota_j == jidx
        knocked = jnp.where(onehot, jnp.inf, cur)
        d2_s[...] = knocked
        return onehot, jidx, jnp.min(knocked, axis=1, keepdims=True)

    def body1(t, m):
        onehot, _, m = knn_step(m)
        ohb = onehot.astype(bf16)
        dn = (((1,), (0,)), ((), ()))
        xj = (lax.dot_general(ohb, xhi, dn, preferred_element_type=f32)
              + lax.dot_general(ohb, xmid, dn, preferred_element_type=f32)
              + lax.dot_general(ohb, xlo, dn, preferred_element_type=f32))
        a = jnp.dot(xj - x, wv_ref[...], preferred_element_type=f32)
        pre = ux_s[...] + a + b1_ref[...]
        bn = pre / jnp.sqrt(1.0 + 1e-5) * g1_ref[...] + be1_ref[...]
        h = jnp.dot(jax.nn.relu(bn), w2_ref[...],
                    preferred_element_type=f32) + b2_ref[...]
        x1_s[...] = jnp.maximum(x1_s[...], h)
        return m

    lax.fori_loop(0, K, body1, jnp.min(d2_s[...], axis=1, keepdims=True))

    x1 = x1_s[...]
    sq2 = jnp.sum(x1 * x1, axis=1, keepdims=True)
    g2 = lax.dot_general(x1, x1, (((1,), (1,)), ((), ())),
                         preferred_element_type=f32)
    sqrow2 = lax.dot_general(ones, sq2, (((1,), (1,)), ((), ())),
                             preferred_element_type=f32, precision=HI)
    d2_s[...] = (sq2 + sqrow2) - 2.0 * g2

    lane_t = lax.broadcasted_iota(jnp.int32, (P, 128), 1)
    boff = pl.program_id(0) * P
    idx_ref[0] = jnp.zeros((P, 128), jnp.int32)

    def body2(t, m):
        onehot, jidx, m = knn_step(m)
        idx_ref[0] = jnp.where(lane_t == t, jidx + boff, idx_ref[0])
        return m

    lax.fori_loop(0, K, body2, jnp.min(d2_s[...], axis=1, keepdims=True))

    d_ref[0] = jnp.dot(x1, wcd_ref[...], preferred_element_type=f32)
    c_ref[0] = jnp.dot(x1, wcc_ref[...], preferred_element_type=f32)
    hp_ref[0] = jnp.dot(x1, wla_ref[...], preferred_element_type=f32)


def _full(shape):
    return pl.BlockSpec(shape, lambda b: (0,) * len(shape))


def _run_a(pos_p, wu8, wv8, b1r, g1r, be1r, w2, b2, wcc, wcd, wla):
    blk = pl.BlockSpec((1, P, 128), lambda b: (b, 0, 0))
    return pl.pallas_call(
        _cloud_body,
        grid=(B,),
        in_specs=[
            pl.BlockSpec((1, P, 8), lambda b: (b, 0, 0)),
            _full((8, 64)), _full((8, 64)),
            _full((1, 64)), _full((1, 64)), _full((1, 64)),
            _full((64, 64)), _full((1, 64)),
            _full((64, 128)), _full((64, 128)), _full((64, 128)),
        ],
        out_specs=[blk, blk, blk, blk],
        out_shape=[
            jax.ShapeDtypeStruct((B, P, 128), jnp.int32),
            jax.ShapeDtypeStruct((B, P, 128), jnp.float32),
            jax.ShapeDtypeStruct((B, P, 128), jnp.float32),
            jax.ShapeDtypeStruct((B, P, 128), jnp.float32),
        ],
        scratch_shapes=[
            pltpu.VMEM((P, P), jnp.float32),
            pltpu.VMEM((P, 64), jnp.float32),
            pltpu.VMEM((P, 64), jnp.float32),
        ],
        compiler_params=pltpu.CompilerParams(
            dimension_semantics=("arbitrary",),
        ),
    )(pos_p, wu8, wv8, b1r, g1r, be1r, w2, b2, wcc, wcd, wla)


_sc_mesh = plsc.VectorSubcoreMesh(core_axis_name="c", subcore_axis_name="s")


@functools.partial(
    pl.kernel,
    mesh=_sc_mesh,
    out_type=jax.ShapeDtypeStruct((BP, 128), jnp.float32),
    scratch_types=[
        pltpu.VMEM((G * K,), jnp.int32),
        pltpu.VMEM((G * K, 128), jnp.float32),
        pltpu.VMEM((G, 128), jnp.float32),
        pltpu.SemaphoreType.DMA,
    ],
)
def _sc_gather_max(d_hbm, idx_hbm, out_hbm, idxbuf, rows, outbuf, sem):
    wid = lax.axis_index("s") * 2 + lax.axis_index("c")
    base_pt = wid * PTS_PER_W

    def grp(gi, carry):
        pt = base_pt + gi * G
        off = pl.multiple_of(pt * K, 8)
        pltpu.sync_copy(idx_hbm.at[pl.ds(off, G * K)], idxbuf)
        pltpu.async_copy(d_hbm.at[idxbuf], rows, sem).wait()
        for p in range(G):
            for c in range(8):
                acc = rows[p * K, pl.ds(c * 16, 16)]
                for r in range(1, K):
                    acc = jnp.maximum(acc, rows[p * K + r, pl.ds(c * 16, 16)])
                outbuf[p, pl.ds(c * 16, 16)] = acc
        pltpu.sync_copy(outbuf, out_hbm.at[pl.ds(pt, G)])
        return carry

    lax.fori_loop(0, PTS_PER_W // G, grp, 0)


def _head_body(hp_ref, c_ref, md_ref, wlb_ref, bc2_ref, bl_ref, out_ref):
    f32 = jnp.float32
    x2 = (c_ref[0] + md_ref[0]) + bc2_ref[...]
    h = (hp_ref[0] + jnp.dot(x2, wlb_ref[...], preferred_element_type=f32)
         ) + bl_ref[...]
    out_ref[0] = jnp.max(h, axis=0, keepdims=True)


def _run_b(hp, cpart, maxd, wlb, bc2r, blr):
    blk = pl.BlockSpec((1, P, 128), lambda b: (b, 0, 0))
    return pl.pallas_call(
        _head_body,
        grid=(B,),
        in_specs=[blk, blk, blk,
                  _full((128, 128)), _full((1, 128)), _full((1, 128))],
        out_specs=pl.BlockSpec((1, 1, 128), lambda b: (b, 0, 0)),
        out_shape=jax.ShapeDtypeStruct((B, 1, 128), jnp.float32),
        compiler_params=pltpu.CompilerParams(
            dimension_semantics=("arbitrary",),
        ),
    )(hp, cpart, maxd, wlb, bc2r, blr)


def kernel(pos, batch, W1a, b1a, g1a, be1a, W2a, b2a, Wc2, bc2, Wl, bl):
    f32 = jnp.float32
    wu8 = jnp.zeros((8, 64), f32).at[:3].set(W1a[:3])
    wv8 = jnp.zeros((8, 64), f32).at[:3].set(W1a[3:6])

    wcc = Wc2[:64] - Wc2[64:]
    wcd = Wc2[64:]

    pos_p = jnp.zeros((B, P, 8), f32).at[:, :, :3].set(pos.reshape(B, P, 3))

    idx_all, d_all, cpart, hp = _run_a(
        pos_p, wu8, wv8,
        b1a.reshape(1, 64), g1a.reshape(1, 64), be1a.reshape(1, 64),
        W2a, b2a.reshape(1, 64), wcc, wcd, Wl[:64])

    idx_flat = idx_all.reshape(BP, 128)[:, :K].reshape(-1)
    maxd = _sc_gather_max(d_all.reshape(BP, 128), idx_flat)

    out = _run_b(hp, cpart, maxd.reshape(B, P, 128),
                 Wl[64:], bc2.reshape(1, 128), bl.reshape(1, 128))
    return out.reshape(B, 128)


# speedup vs baseline: 1.2946x; 1.0863x over previous
import functools

import jax
import jax.numpy as jnp
from jax import lax
from jax.experimental import pallas as pl
from jax.experimental.pallas import tpu as pltpu
from jax.experimental.pallas import tpu_sc as plsc

B = 16
P = 1024
K = 20
BP = B * P
G = 4
PTS_PER_W = BP // 32


def _cloud_body(pos_ref, wu_ref, wv_ref, b1_ref, g1_ref, be1_ref,
                w2_ref, b2_ref, wcc_ref, wcd_ref, wla_ref,
                idx_ref, d_ref, c_ref, hp_ref,
                d2_s, ux_s, x1_s):
    f32 = jnp.float32
    HI = lax.Precision.HIGHEST
    x = pos_ref[0]
    sq = jnp.sum(x * x, axis=1, keepdims=True)
    ones = jnp.ones((P, 1), f32)

    g = lax.dot_general(x, x, (((1,), (1,)), ((), ())),
                        preferred_element_type=f32)
    sqrow = lax.dot_general(ones, sq, (((1,), (1,)), ((), ())),
                            preferred_element_type=f32, precision=HI)
    d2_s[...] = (sq + sqrow) - 2.0 * g

    ux_s[...] = jnp.dot(x, wu_ref[...], preferred_element_type=f32)
    x1_s[...] = jnp.full((P, 64), -jnp.inf, f32)

    bf16 = jnp.bfloat16
    xhi = x.astype(bf16)
    r1 = x - xhi.astype(f32)
    xmid = r1.astype(bf16)
    xlo = (r1 - xmid.astype(f32)).astype(bf16)

    iota_j = lax.broadcasted_iota(jnp.int32, (P, P), 1)

    def knn_step(m):
        cur = d2_s[...]
        jidx = jnp.min(jnp.where(cur == m, iota_j, P), axis=1,
                       keepdims=True)
        onehot = iota_j == jidx
        knocked = jnp.where(onehot, jnp.inf, cur)
        d2_s[...] = knocked
        return onehot, jidx, jnp.min(knocked, axis=1, keepdims=True)

    def body1(t, m):
        onehot, _, m = knn_step(m)
        ohb = onehot.astype(bf16)
        dn = (((1,), (0,)), ((), ()))
        xj = (lax.dot_general(ohb, xhi, dn, preferred_element_type=f32)
              + lax.dot_general(ohb, xmid, dn, preferred_element_type=f32)
              + lax.dot_general(ohb, xlo, dn, preferred_element_type=f32))
        a = jnp.dot(xj - x, wv_ref[...], preferred_element_type=f32)
        pre = ux_s[...] + a + b1_ref[...]
        bn = pre / jnp.sqrt(1.0 + 1e-5) * g1_ref[...] + be1_ref[...]
        h = jnp.dot(jax.nn.relu(bn), w2_ref[...],
                    preferred_element_type=f32) + b2_ref[...]
        x1_s[...] = jnp.maximum(x1_s[...], h)
        return m

    lax.fori_loop(0, K, body1, jnp.min(d2_s[...], axis=1, keepdims=True))

    x1 = x1_s[...]
    sq2 = jnp.sum(x1 * x1, axis=1, keepdims=True)
    g2 = lax.dot_general(x1, x1, (((1,), (1,)), ((), ())),
                         preferred_element_type=f32)
    sqrow2 = lax.dot_general(ones, sq2, (((1,), (1,)), ((), ())),
                             preferred_element_type=f32, precision=HI)
    d2_s[...] = (sq2 + sqrow2) - 2.0 * g2

    lane_t = lax.broadcasted_iota(jnp.int32, (P, 128), 1)
    boff = pl.program_id(0) * P
    idx_ref[0] = jnp.zeros((P, 128), jnp.int32)

    def body2(t, m):
        onehot, jidx, m = knn_step(m)
        idx_ref[0] = jnp.where(lane_t == t, jidx + boff, idx_ref[0])
        return m

    lax.fori_loop(0, K, body2, jnp.min(d2_s[...], axis=1, keepdims=True))

    d_ref[0] = jnp.dot(x1, wcd_ref[...], preferred_element_type=f32)
    c_ref[0] = jnp.dot(x1, wcc_ref[...], preferred_element_type=f32)
    hp_ref[0] = jnp.dot(x1, wla_ref[...], preferred_element_type=f32)


def _full(shape):
    return pl.BlockSpec(shape, lambda b: (0,) * len(shape))


def _run_a(pos_p, wu8, wv8, b1r, g1r, be1r, w2, b2, wcc, wcd, wla):
    nb = pos_p.shape[0]
    blk = pl.BlockSpec((1, P, 128), lambda b: (b, 0, 0))
    return pl.pallas_call(
        _cloud_body,
        grid=(nb,),
        in_specs=[
            pl.BlockSpec((1, P, 8), lambda b: (b, 0, 0)),
            _full((8, 64)), _full((8, 64)),
            _full((1, 64)), _full((1, 64)), _full((1, 64)),
            _full((64, 64)), _full((1, 64)),
            _full((64, 128)), _full((64, 128)), _full((64, 128)),
        ],
        out_specs=[blk, blk, blk, blk],
        out_shape=[
            jax.ShapeDtypeStruct((nb, P, 128), jnp.int32),
            jax.ShapeDtypeStruct((nb, P, 128), jnp.float32),
            jax.ShapeDtypeStruct((nb, P, 128), jnp.float32),
            jax.ShapeDtypeStruct((nb, P, 128), jnp.float32),
        ],
        scratch_shapes=[
            pltpu.VMEM((P, P), jnp.float32),
            pltpu.VMEM((P, 64), jnp.float32),
            pltpu.VMEM((P, 64), jnp.float32),
        ],
        compiler_params=pltpu.CompilerParams(
            dimension_semantics=("arbitrary",),
        ),
    )(pos_p, wu8, wv8, b1r, g1r, be1r, w2, b2, wcc, wcd, wla)


_sc_mesh = plsc.VectorSubcoreMesh(core_axis_name="c", subcore_axis_name="s")


def _make_sc_gather_max(bp):
    ppw = bp // 32

    @functools.partial(
        pl.kernel,
        mesh=_sc_mesh,
        out_type=jax.ShapeDtypeStruct((bp, 128), jnp.float32),
        scratch_types=[
            pltpu.VMEM((G * K,), jnp.int32),
            pltpu.VMEM((G * K, 128), jnp.float32),
            pltpu.VMEM((G, 128), jnp.float32),
            pltpu.SemaphoreType.DMA,
        ],
    )
    def _sc_gather_max(d_hbm, idx_hbm, out_hbm, idxbuf, rows, outbuf, sem):
        wid = lax.axis_index("s") * 2 + lax.axis_index("c")
        base_pt = wid * ppw

        def grp(gi, carry):
            pt = base_pt + gi * G
            off = pl.multiple_of(pt * K, 8)
            pltpu.sync_copy(idx_hbm.at[pl.ds(off, G * K)], idxbuf)
            pltpu.async_copy(d_hbm.at[idxbuf], rows, sem).wait()
            for p in range(G):
                for c in range(8):
                    acc = rows[p * K, pl.ds(c * 16, 16)]
                    for r in range(1, K):
                        acc = jnp.maximum(acc,
                                          rows[p * K + r, pl.ds(c * 16, 16)])
                    outbuf[p, pl.ds(c * 16, 16)] = acc
            pltpu.sync_copy(outbuf, out_hbm.at[pl.ds(pt, G)])
            return carry

        lax.fori_loop(0, ppw // G, grp, 0)

    return _sc_gather_max


_N_CHUNKS = 2
_SC_CHUNK = _make_sc_gather_max(BP // _N_CHUNKS)


def _head_body(hp_ref, c_ref, md_ref, wlb_ref, bc2_ref, bl_ref, out_ref):
    f32 = jnp.float32
    x2 = (c_ref[0] + md_ref[0]) + bc2_ref[...]
    h = (hp_ref[0] + jnp.dot(x2, wlb_ref[...], preferred_element_type=f32)
         ) + bl_ref[...]
    out_ref[0] = jnp.max(h, axis=0, keepdims=True)


def _run_b(hp, cpart, maxd, wlb, bc2r, blr):
    blk = pl.BlockSpec((1, P, 128), lambda b: (b, 0, 0))
    return pl.pallas_call(
        _head_body,
        grid=(B,),
        in_specs=[blk, blk, blk,
                  _full((128, 128)), _full((1, 128)), _full((1, 128))],
        out_specs=pl.BlockSpec((1, 1, 128), lambda b: (b, 0, 0)),
        out_shape=jax.ShapeDtypeStruct((B, 1, 128), jnp.float32),
        compiler_params=pltpu.CompilerParams(
            dimension_semantics=("arbitrary",),
        ),
    )(hp, cpart, maxd, wlb, bc2r, blr)


def kernel(pos, batch, W1a, b1a, g1a, be1a, W2a, b2a, Wc2, bc2, Wl, bl):
    f32 = jnp.float32
    wu8 = jnp.zeros((8, 64), f32).at[:3].set(W1a[:3])
    wv8 = jnp.zeros((8, 64), f32).at[:3].set(W1a[3:6])

    wcc = Wc2[:64] - Wc2[64:]
    wcd = Wc2[64:]

    pos_p = jnp.zeros((B, P, 8), f32).at[:, :, :3].set(pos.reshape(B, P, 3))

    bc = B // _N_CHUNKS
    maxds, cparts, hps = [], [], []
    for ci in range(_N_CHUNKS):
        idx_c, d_c, c_c, hp_c = _run_a(
            pos_p[ci * bc:(ci + 1) * bc], wu8, wv8,
            b1a.reshape(1, 64), g1a.reshape(1, 64), be1a.reshape(1, 64),
            W2a, b2a.reshape(1, 64), wcc, wcd, Wl[:64])
        idx_flat = idx_c.reshape(bc * P, 128)[:, :K].reshape(-1)
        maxds.append(_SC_CHUNK(d_c.reshape(bc * P, 128), idx_flat))
        cparts.append(c_c)
        hps.append(hp_c)

    maxd = jnp.concatenate(maxds).reshape(B, P, 128)
    cpart = jnp.concatenate(cparts)
    hp = jnp.concatenate(hps)

    out = _run_b(hp, cpart, maxd,
                 Wl[64:], bc2.reshape(1, 128), bl.reshape(1, 128))
    return out.reshape(B, 128)


# speedup vs baseline: 1.3404x; 1.0354x over previous
import functools

import jax
import jax.numpy as jnp
from jax import lax
from jax.experimental import pallas as pl
from jax.experimental.pallas import tpu as pltpu
from jax.experimental.pallas import tpu_sc as plsc

B = 16
P = 1024
K = 20
BP = B * P
G = 4
PTS_PER_W = BP // 32


def _cloud_body(pos_ref, wu_ref, wv_ref, b1_ref, g1_ref, be1_ref,
                w2_ref, b2_ref, wcc_ref, wcd_ref, wla_ref,
                idx_ref, d_ref, c_ref, hp_ref,
                d2_s, ux_s, x1_s):
    f32 = jnp.float32
    HI = lax.Precision.HIGHEST
    x = pos_ref[0]
    sq = jnp.sum(x * x, axis=1, keepdims=True)
    ones = jnp.ones((P, 1), f32)

    g = lax.dot_general(x, x, (((1,), (1,)), ((), ())),
                        preferred_element_type=f32)
    sqrow = lax.dot_general(ones, sq, (((1,), (1,)), ((), ())),
                            preferred_element_type=f32, precision=HI)
    d2_s[...] = (sq + sqrow) - 2.0 * g

    ux_s[...] = jnp.dot(x, wu_ref[...], preferred_element_type=f32)
    x1_s[...] = jnp.full((P, 64), -jnp.inf, f32)

    bf16 = jnp.bfloat16
    xhi = x.astype(bf16)
    r1 = x - xhi.astype(f32)
    xmid = r1.astype(bf16)
    xlo = (r1 - xmid.astype(f32)).astype(bf16)

    iota_j = lax.broadcasted_iota(jnp.int32, (P, P), 1)

    def knn_step(m):
        cur = d2_s[...]
        jidx = jnp.min(jnp.where(cur == m, iota_j, P), axis=1,
                       keepdims=True)
        onehot = iota_j == jidx
        knocked = jnp.where(onehot, jnp.inf, cur)
        d2_s[...] = knocked
        return onehot, jidx, jnp.min(knocked, axis=1, keepdims=True)

    def body1(t, m):
        onehot, _, m = knn_step(m)
        ohb = onehot.astype(bf16)
        dn = (((1,), (0,)), ((), ()))
        xj = (lax.dot_general(ohb, xhi, dn, preferred_element_type=f32)
              + lax.dot_general(ohb, xmid, dn, preferred_element_type=f32)
              + lax.dot_general(ohb, xlo, dn, preferred_element_type=f32))
        a = jnp.dot(xj - x, wv_ref[...], preferred_element_type=f32)
        pre = ux_s[...] + a + b1_ref[...]
        bn = pre / jnp.sqrt(1.0 + 1e-5) * g1_ref[...] + be1_ref[...]
        h = jnp.dot(jax.nn.relu(bn), w2_ref[...],
                    preferred_element_type=f32) + b2_ref[...]
        x1_s[...] = jnp.maximum(x1_s[...], h)
        return m

    lax.fori_loop(0, K, body1, jnp.min(d2_s[...], axis=1, keepdims=True))

    x1 = x1_s[...]
    sq2 = jnp.sum(x1 * x1, axis=1, keepdims=True)
    g2 = lax.dot_general(x1, x1, (((1,), (1,)), ((), ())),
                         preferred_element_type=f32)
    sqrow2 = lax.dot_general(ones, sq2, (((1,), (1,)), ((), ())),
                             preferred_element_type=f32, precision=HI)
    d2_s[...] = (sq2 + sqrow2) - 2.0 * g2

    lane_t = lax.broadcasted_iota(jnp.int32, (P, 128), 1)
    boff = pl.program_id(0) * P
    idx_ref[0] = jnp.zeros((P, 128), jnp.int32)

    def body2(t, m):
        onehot, jidx, m = knn_step(m)
        idx_ref[0] = jnp.where(lane_t == t, jidx + boff, idx_ref[0])
        return m

    lax.fori_loop(0, K, body2, jnp.min(d2_s[...], axis=1, keepdims=True))

    d_ref[0] = jnp.dot(x1, wcd_ref[...], preferred_element_type=f32)
    c_ref[0] = jnp.dot(x1, wcc_ref[...], preferred_element_type=f32)
    hp_ref[0] = jnp.dot(x1, wla_ref[...], preferred_element_type=f32)


def _full(shape):
    return pl.BlockSpec(shape, lambda b: (0,) * len(shape))


def _run_a(pos_p, wu8, wv8, b1r, g1r, be1r, w2, b2, wcc, wcd, wla):
    nb = pos_p.shape[0]
    blk = pl.BlockSpec((1, P, 128), lambda b: (b, 0, 0))
    return pl.pallas_call(
        _cloud_body,
        grid=(nb,),
        in_specs=[
            pl.BlockSpec((1, P, 8), lambda b: (b, 0, 0)),
            _full((8, 64)), _full((8, 64)),
            _full((1, 64)), _full((1, 64)), _full((1, 64)),
            _full((64, 64)), _full((1, 64)),
            _full((64, 128)), _full((64, 128)), _full((64, 128)),
        ],
        out_specs=[blk, blk, blk, blk],
        out_shape=[
            jax.ShapeDtypeStruct((nb, P, 128), jnp.int32),
            jax.ShapeDtypeStruct((nb, P, 128), jnp.float32),
            jax.ShapeDtypeStruct((nb, P, 128), jnp.float32),
            jax.ShapeDtypeStruct((nb, P, 128), jnp.float32),
        ],
        scratch_shapes=[
            pltpu.VMEM((P, P), jnp.float32),
            pltpu.VMEM((P, 64), jnp.float32),
            pltpu.VMEM((P, 64), jnp.float32),
        ],
        compiler_params=pltpu.CompilerParams(
            dimension_semantics=("arbitrary",),
        ),
    )(pos_p, wu8, wv8, b1r, g1r, be1r, w2, b2, wcc, wcd, wla)


_sc_mesh = plsc.VectorSubcoreMesh(core_axis_name="c", subcore_axis_name="s")


def _make_sc_gather_max(bp):
    ppw = bp // 32

    @functools.partial(
        pl.kernel,
        mesh=_sc_mesh,
        out_type=jax.ShapeDtypeStruct((bp, 128), jnp.float32),
        scratch_types=[
            pltpu.VMEM((G * K,), jnp.int32),
            pltpu.VMEM((G * K, 128), jnp.float32),
            pltpu.VMEM((G, 128), jnp.float32),
            pltpu.SemaphoreType.DMA,
        ],
    )
    def _sc_gather_max(d_hbm, idx_hbm, out_hbm, idxbuf, rows, outbuf, sem):
        wid = lax.axis_index("s") * 2 + lax.axis_index("c")
        base_pt = wid * ppw

        def grp(gi, carry):
            pt = base_pt + gi * G
            off = pl.multiple_of(pt * K, 8)
            pltpu.sync_copy(idx_hbm.at[pl.ds(off, G * K)], idxbuf)
            pltpu.async_copy(d_hbm.at[idxbuf], rows, sem).wait()
            for p in range(G):
                for c in range(8):
                    acc = rows[p * K, pl.ds(c * 16, 16)]
                    for r in range(1, K):
                        acc = jnp.maximum(acc,
                                          rows[p * K + r, pl.ds(c * 16, 16)])
                    outbuf[p, pl.ds(c * 16, 16)] = acc
            pltpu.sync_copy(outbuf, out_hbm.at[pl.ds(pt, G)])
            return carry

        lax.fori_loop(0, ppw // G, grp, 0)

    return _sc_gather_max


_N_CHUNKS = 4
_SC_CHUNK = _make_sc_gather_max(BP // _N_CHUNKS)


def _head_body(hp_ref, c_ref, md_ref, wlb_ref, bc2_ref, bl_ref, out_ref):
    f32 = jnp.float32
    x2 = (c_ref[0] + md_ref[0]) + bc2_ref[...]
    h = (hp_ref[0] + jnp.dot(x2, wlb_ref[...], preferred_element_type=f32)
         ) + bl_ref[...]
    out_ref[0] = jnp.max(h, axis=0, keepdims=True)


def _run_b(hp, cpart, maxd, wlb, bc2r, blr):
    blk = pl.BlockSpec((1, P, 128), lambda b: (b, 0, 0))
    return pl.pallas_call(
        _head_body,
        grid=(B,),
        in_specs=[blk, blk, blk,
                  _full((128, 128)), _full((1, 128)), _full((1, 128))],
        out_specs=pl.BlockSpec((1, 1, 128), lambda b: (b, 0, 0)),
        out_shape=jax.ShapeDtypeStruct((B, 1, 128), jnp.float32),
        compiler_params=pltpu.CompilerParams(
            dimension_semantics=("arbitrary",),
        ),
    )(hp, cpart, maxd, wlb, bc2r, blr)


def kernel(pos, batch, W1a, b1a, g1a, be1a, W2a, b2a, Wc2, bc2, Wl, bl):
    f32 = jnp.float32
    wu8 = jnp.zeros((8, 64), f32).at[:3].set(W1a[:3])
    wv8 = jnp.zeros((8, 64), f32).at[:3].set(W1a[3:6])

    wcc = Wc2[:64] - Wc2[64:]
    wcd = Wc2[64:]

    pos_p = jnp.zeros((B, P, 8), f32).at[:, :, :3].set(pos.reshape(B, P, 3))

    bc = B // _N_CHUNKS
    maxds, cparts, hps = [], [], []
    for ci in range(_N_CHUNKS):
        idx_c, d_c, c_c, hp_c = _run_a(
            pos_p[ci * bc:(ci + 1) * bc], wu8, wv8,
            b1a.reshape(1, 64), g1a.reshape(1, 64), be1a.reshape(1, 64),
            W2a, b2a.reshape(1, 64), wcc, wcd, Wl[:64])
        idx_flat = idx_c.reshape(bc * P, 128)[:, :K].reshape(-1)
        maxds.append(_SC_CHUNK(d_c.reshape(bc * P, 128), idx_flat))
        cparts.append(c_c)
        hps.append(hp_c)

    maxd = jnp.concatenate(maxds).reshape(B, P, 128)
    cpart = jnp.concatenate(cparts)
    hp = jnp.concatenate(hps)

    out = _run_b(hp, cpart, maxd,
                 Wl[64:], bc2.reshape(1, 128), bl.reshape(1, 128))
    return out.reshape(B, 128)


# speedup vs baseline: 1.3688x; 1.0211x over previous
import functools

import jax
import jax.numpy as jnp
from jax import lax
from jax.experimental import pallas as pl
from jax.experimental.pallas import tpu as pltpu
from jax.experimental.pallas import tpu_sc as plsc

B = 16
P = 1024
K = 20
BP = B * P
G = 4
PTS_PER_W = BP // 32


def _cloud_body(pos_ref, wu_ref, wv_ref, b1_ref, g1_ref, be1_ref,
                w2_ref, b2_ref, wcc_ref, wcd_ref, wla_ref,
                idx_ref, d_ref, c_ref, hp_ref,
                d2_s, ux_s, x1_s):
    f32 = jnp.float32
    HI = lax.Precision.HIGHEST
    x = pos_ref[0]
    sq = jnp.sum(x * x, axis=1, keepdims=True)
    ones = jnp.ones((P, 1), f32)

    g = lax.dot_general(x, x, (((1,), (1,)), ((), ())),
                        preferred_element_type=f32)
    sqrow = lax.dot_general(ones, sq, (((1,), (1,)), ((), ())),
                            preferred_element_type=f32, precision=HI)
    d2_s[...] = (sq + sqrow) - 2.0 * g

    ux_s[...] = jnp.dot(x, wu_ref[...], preferred_element_type=f32)
    x1_s[...] = jnp.full((P, 64), -jnp.inf, f32)

    bf16 = jnp.bfloat16
    xhi = x.astype(bf16)
    r1 = x - xhi.astype(f32)
    xmid = r1.astype(bf16)
    xlo = (r1 - xmid.astype(f32)).astype(bf16)

    iota_j = lax.broadcasted_iota(jnp.int32, (P, P), 1)

    def knn_step(m):
        cur = d2_s[...]
        jidx = jnp.min(jnp.where(cur == m, iota_j, P), axis=1,
                       keepdims=True)
        onehot = iota_j == jidx
        knocked = jnp.where(onehot, jnp.inf, cur)
        d2_s[...] = knocked
        return onehot, jidx, jnp.min(knocked, axis=1, keepdims=True)

    def body1(t, m):
        onehot, _, m = knn_step(m)
        ohb = onehot.astype(bf16)
        dn = (((1,), (0,)), ((), ()))
        xj = (lax.dot_general(ohb, xhi, dn, preferred_element_type=f32)
              + lax.dot_general(ohb, xmid, dn, preferred_element_type=f32)
              + lax.dot_general(ohb, xlo, dn, preferred_element_type=f32))
        a = jnp.dot(xj - x, wv_ref[...], preferred_element_type=f32)
        pre = ux_s[...] + a + b1_ref[...]
        bn = pre / jnp.sqrt(1.0 + 1e-5) * g1_ref[...] + be1_ref[...]
        h = jnp.dot(jax.nn.relu(bn), w2_ref[...],
                    preferred_element_type=f32) + b2_ref[...]
        x1_s[...] = jnp.maximum(x1_s[...], h)
        return m

    lax.fori_loop(0, K, body1, jnp.min(d2_s[...], axis=1, keepdims=True))

    x1 = x1_s[...]
    sq2 = jnp.sum(x1 * x1, axis=1, keepdims=True)
    g2 = lax.dot_general(x1, x1, (((1,), (1,)), ((), ())),
                         preferred_element_type=f32)
    sqrow2 = lax.dot_general(ones, sq2, (((1,), (1,)), ((), ())),
                             preferred_element_type=f32, precision=HI)
    d2_s[...] = (sq2 + sqrow2) - 2.0 * g2

    lane_t = lax.broadcasted_iota(jnp.int32, (P, 128), 1)
    boff = pl.program_id(0) * P
    idx_ref[0] = jnp.zeros((P, 128), jnp.int32)

    def body2(t, m):
        onehot, jidx, m = knn_step(m)
        idx_ref[0] = jnp.where(lane_t == t, jidx + boff, idx_ref[0])
        return m

    lax.fori_loop(0, K, body2, jnp.min(d2_s[...], axis=1, keepdims=True))

    d_ref[0] = jnp.dot(x1, wcd_ref[...], preferred_element_type=f32)
    c_ref[0] = jnp.dot(x1, wcc_ref[...], preferred_element_type=f32)
    hp_ref[0] = jnp.dot(x1, wla_ref[...], preferred_element_type=f32)


def _full(shape):
    return pl.BlockSpec(shape, lambda b: (0,) * len(shape))


def _run_a(pos_p, wu8, wv8, b1r, g1r, be1r, w2, b2, wcc, wcd, wla):
    nb = pos_p.shape[0]
    blk = pl.BlockSpec((1, P, 128), lambda b: (b, 0, 0))
    return pl.pallas_call(
        _cloud_body,
        grid=(nb,),
        in_specs=[
            pl.BlockSpec((1, P, 8), lambda b: (b, 0, 0)),
            _full((8, 64)), _full((8, 64)),
            _full((1, 64)), _full((1, 64)), _full((1, 64)),
            _full((64, 64)), _full((1, 64)),
            _full((64, 128)), _full((64, 128)), _full((64, 128)),
        ],
        out_specs=[blk, blk, blk, blk],
        out_shape=[
            jax.ShapeDtypeStruct((nb, P, 128), jnp.int32),
            jax.ShapeDtypeStruct((nb, P, 128), jnp.float32),
            jax.ShapeDtypeStruct((nb, P, 128), jnp.float32),
            jax.ShapeDtypeStruct((nb, P, 128), jnp.float32),
        ],
        scratch_shapes=[
            pltpu.VMEM((P, P), jnp.float32),
            pltpu.VMEM((P, 64), jnp.float32),
            pltpu.VMEM((P, 64), jnp.float32),
        ],
        compiler_params=pltpu.CompilerParams(
            dimension_semantics=("arbitrary",),
        ),
    )(pos_p, wu8, wv8, b1r, g1r, be1r, w2, b2, wcc, wcd, wla)


_sc_mesh = plsc.VectorSubcoreMesh(core_axis_name="c", subcore_axis_name="s")


def _make_sc_gather_max(bp):
    ppw = bp // 32

    @functools.partial(
        pl.kernel,
        mesh=_sc_mesh,
        out_type=jax.ShapeDtypeStruct((bp, 128), jnp.float32),
        scratch_types=[
            pltpu.VMEM((G * K,), jnp.int32),
            pltpu.VMEM((G * K, 128), jnp.float32),
            pltpu.VMEM((G, 128), jnp.float32),
            pltpu.SemaphoreType.DMA,
        ],
    )
    def _sc_gather_max(d_hbm, idx_hbm, out_hbm, idxbuf, rows, outbuf, sem):
        wid = lax.axis_index("s") * 2 + lax.axis_index("c")
        base_pt = wid * ppw

        def grp(gi, carry):
            pt = base_pt + gi * G
            off = pl.multiple_of(pt * K, 8)
            pltpu.sync_copy(idx_hbm.at[pl.ds(off, G * K)], idxbuf)
            pltpu.async_copy(d_hbm.at[idxbuf], rows, sem).wait()
            for p in range(G):
                for c in range(8):
                    acc = rows[p * K, pl.ds(c * 16, 16)]
                    for r in range(1, K):
                        acc = jnp.maximum(acc,
                                          rows[p * K + r, pl.ds(c * 16, 16)])
                    outbuf[p, pl.ds(c * 16, 16)] = acc
            pltpu.sync_copy(outbuf, out_hbm.at[pl.ds(pt, G)])
            return carry

        lax.fori_loop(0, ppw // G, grp, 0)

    return _sc_gather_max


_N_CHUNKS = 8
_SC_CHUNK = _make_sc_gather_max(BP // _N_CHUNKS)


def _head_body(hp_ref, c_ref, md_ref, wlb_ref, bc2_ref, bl_ref, out_ref):
    f32 = jnp.float32
    x2 = (c_ref[0] + md_ref[0]) + bc2_ref[...]
    h = (hp_ref[0] + jnp.dot(x2, wlb_ref[...], preferred_element_type=f32)
         ) + bl_ref[...]
    out_ref[0] = jnp.max(h, axis=0, keepdims=True)


def _run_b(hp, cpart, maxd, wlb, bc2r, blr):
    blk = pl.BlockSpec((1, P, 128), lambda b: (b, 0, 0))
    return pl.pallas_call(
        _head_body,
        grid=(B,),
        in_specs=[blk, blk, blk,
                  _full((128, 128)), _full((1, 128)), _full((1, 128))],
        out_specs=pl.BlockSpec((1, 1, 128), lambda b: (b, 0, 0)),
        out_shape=jax.ShapeDtypeStruct((B, 1, 128), jnp.float32),
        compiler_params=pltpu.CompilerParams(
            dimension_semantics=("arbitrary",),
        ),
    )(hp, cpart, maxd, wlb, bc2r, blr)


def kernel(pos, batch, W1a, b1a, g1a, be1a, W2a, b2a, Wc2, bc2, Wl, bl):
    f32 = jnp.float32
    wu8 = jnp.zeros((8, 64), f32).at[:3].set(W1a[:3])
    wv8 = jnp.zeros((8, 64), f32).at[:3].set(W1a[3:6])

    wcc = Wc2[:64] - Wc2[64:]
    wcd = Wc2[64:]

    pos_p = jnp.zeros((B, P, 8), f32).at[:, :, :3].set(pos.reshape(B, P, 3))

    bc = B // _N_CHUNKS
    maxds, cparts, hps = [], [], []
    for ci in range(_N_CHUNKS):
        idx_c, d_c, c_c, hp_c = _run_a(
            pos_p[ci * bc:(ci + 1) * bc], wu8, wv8,
            b1a.reshape(1, 64), g1a.reshape(1, 64), be1a.reshape(1, 64),
            W2a, b2a.reshape(1, 64), wcc, wcd, Wl[:64])
        idx_flat = idx_c.reshape(bc * P, 128)[:, :K].reshape(-1)
        maxds.append(_SC_CHUNK(d_c.reshape(bc * P, 128), idx_flat))
        cparts.append(c_c)
        hps.append(hp_c)

    maxd = jnp.concatenate(maxds).reshape(B, P, 128)
    cpart = jnp.concatenate(cparts)
    hp = jnp.concatenate(hps)

    out = _run_b(hp, cpart, maxd,
                 Wl[64:], bc2.reshape(1, 128), bl.reshape(1, 128))
    return out.reshape(B, 128)
